# baseline (speedup 1.0000x reference)
import jax
import jax.numpy as jnp
from jax.experimental import pallas as pl

B = 256
ROWS = 1000
NEG = -1e30
IMAX = 2147483647


def _mlp_body(e_ref, w1_ref, b1_ref, w2_ref, b2_ref, w3_ref, b3_ref, out_ref):
    eb = e_ref[...].astype(jnp.bfloat16)
    h = jnp.maximum(
        jnp.dot(eb, w1_ref[...], preferred_element_type=jnp.float32) + b1_ref[...], 0.0
    )
    h = jnp.maximum(
        jnp.dot(h, w2_ref[...], preferred_element_type=jnp.float32) + b2_ref[...], 0.0
    )
    out_ref[...] = (
        jnp.dot(h, w3_ref[...], preferred_element_type=jnp.float32) + b3_ref[...]
    )


def _segment_body(lg_ref, seg_ref, u_ref, off_ref, lp_ref, act_ref):
    n = lg_ref.shape[0]
    seg = seg_ref[...]
    lg = lg_ref[...]
    ids = jax.lax.broadcasted_iota(jnp.int32, (n, B), 1)
    mask = seg == ids

    segmax = jnp.max(jnp.where(mask, lg, NEG), axis=0, keepdims=True)
    gmax_node = jnp.sum(jnp.where(mask, segmax, 0.0), axis=1, keepdims=True)
    shifted = lg - gmax_node

    segsum = jnp.sum(jnp.where(mask, jnp.exp(shifted), 0.0), axis=0, keepdims=True)
    logz = jnp.log(segsum)
    logz_node = jnp.sum(jnp.where(mask, logz, 0.0), axis=1, keepdims=True)
    lp = shifted - logz_node
    lp_ref[...] = lp

    gum = -jnp.log(-jnp.log(u_ref[...]))
    gl = lp + gum
    gmax2 = jnp.max(jnp.where(mask, gl, NEG), axis=0, keepdims=True)
    idxs = jax.lax.broadcasted_iota(jnp.int32, (n, B), 0)
    cand = jnp.where(mask & (gl == gmax2), idxs, jnp.int32(IMAX))
    arg = jnp.min(cand, axis=0, keepdims=True)
    act_ref[...] = arg - off_ref[...]


def kernel(e, u, batch_non_omni, act_offsets, W1, b1, W2, b2, W3, b3):
    n, k = e.shape
    h = W1.shape[1]

    logits2 = pl.pallas_call(
        _mlp_body,
        grid=(n // ROWS,),
        in_specs=[
            pl.BlockSpec((ROWS, k), lambda i: (i, 0)),
            pl.BlockSpec((k, h), lambda i: (0, 0)),
            pl.BlockSpec((1, h), lambda i: (0, 0)),
            pl.BlockSpec((h, h), lambda i: (0, 0)),
            pl.BlockSpec((1, h), lambda i: (0, 0)),
            pl.BlockSpec((h, 1), lambda i: (0, 0)),
            pl.BlockSpec((1, 1), lambda i: (0, 0)),
        ],
        out_specs=pl.BlockSpec((ROWS, 1), lambda i: (i, 0)),
        out_shape=jax.ShapeDtypeStruct((n, 1), jnp.float32),
    )(
        e,
        W1.astype(jnp.bfloat16),
        b1.reshape(1, h),
        W2,
        b2.reshape(1, h),
        W3,
        b3.reshape(1, 1),
    )

    lp2, act2 = pl.pallas_call(
        _segment_body,
        in_specs=[
            pl.BlockSpec((n, 1), lambda: (0, 0)),
            pl.BlockSpec((n, 1), lambda: (0, 0)),
            pl.BlockSpec((n, 1), lambda: (0, 0)),
            pl.BlockSpec((1, B), lambda: (0, 0)),
        ],
        out_specs=[
            pl.BlockSpec((n, 1), lambda: (0, 0)),
            pl.BlockSpec((1, B), lambda: (0, 0)),
        ],
        out_shape=[
            jax.ShapeDtypeStruct((n, 1), jnp.float32),
            jax.ShapeDtypeStruct((1, B), jnp.int32),
        ],
    )(
        logits2,
        batch_non_omni.reshape(n, 1),
        u.reshape(n, 1),
        act_offsets.reshape(1, B),
    )

    return (logits2.reshape(n), lp2.reshape(n), act2.reshape(B))


# speedup vs baseline: 1.3281x; 1.3281x over previous
import jax
import jax.numpy as jnp
from jax.experimental import pallas as pl

B = 256
NP = 10240
ROWS = 1000
NEG = -1e30
IMAX = 2147483647


def _mlp_body(e_ref, w1_ref, b1_ref, w2_ref, b2_ref, w3_ref, b3_ref, out_ref):
    eb = e_ref[...].astype(jnp.bfloat16)
    h = jnp.maximum(
        jnp.dot(eb, w1_ref[...], preferred_element_type=jnp.float32) + b1_ref[...], 0.0
    )
    h = jnp.maximum(
        jnp.dot(h, w2_ref[...], preferred_element_type=jnp.float32) + b2_ref[...], 0.0
    )
    out_ref[...] = (
        jnp.dot(h, w3_ref[...], preferred_element_type=jnp.float32) + b3_ref[...]
    )


def _segment_body(lg_ref, seg_ref, u_ref, off_ref, lp_ref, act_ref):
    seg = seg_ref[...]
    lg = lg_ref[...]
    ids = jax.lax.broadcasted_iota(jnp.int32, (B, NP), 0)
    mask = seg == ids

    segmax = jnp.max(jnp.where(mask, lg, NEG), axis=1, keepdims=True)
    gmax_node = jnp.sum(jnp.where(mask, segmax, 0.0), axis=0, keepdims=True)
    shifted = lg - gmax_node

    segsum = jnp.sum(jnp.where(mask, jnp.exp(shifted), 0.0), axis=1, keepdims=True)
    logz = jnp.log(segsum)
    logz_node = jnp.sum(jnp.where(mask, logz, 0.0), axis=0, keepdims=True)
    lp = shifted - logz_node
    lp_ref[...] = lp

    gum = -jnp.log(-jnp.log(u_ref[...]))
    gl = lp + gum
    gmax2 = jnp.max(jnp.where(mask, gl, NEG), axis=1, keepdims=True)
    idxs = jax.lax.broadcasted_iota(jnp.int32, (B, NP), 1)
    cand = jnp.where(mask & (gl == gmax2), idxs, jnp.int32(IMAX))
    arg = jnp.min(cand, axis=1, keepdims=True)
    act_ref[...] = arg - off_ref[...]


def kernel(e, u, batch_non_omni, act_offsets, W1, b1, W2, b2, W3, b3):
    n, k = e.shape
    h = W1.shape[1]

    logits2 = pl.pallas_call(
        _mlp_body,
        grid=(n // ROWS,),
        in_specs=[
            pl.BlockSpec((ROWS, k), lambda i: (i, 0)),
            pl.BlockSpec((k, h), lambda i: (0, 0)),
            pl.BlockSpec((1, h), lambda i: (0, 0)),
            pl.BlockSpec((h, h), lambda i: (0, 0)),
            pl.BlockSpec((1, h), lambda i: (0, 0)),
            pl.BlockSpec((h, 1), lambda i: (0, 0)),
            pl.BlockSpec((1, 1), lambda i: (0, 0)),
        ],
        out_specs=pl.BlockSpec((ROWS, 1), lambda i: (i, 0)),
        out_shape=jax.ShapeDtypeStruct((n, 1), jnp.float32),
    )(
        e,
        W1.astype(jnp.bfloat16),
        b1.reshape(1, h),
        W2,
        b2.reshape(1, h),
        W3,
        b3.reshape(1, 1),
    )

    logits = logits2.reshape(n)

    pad = NP - n
    lg_p = jnp.concatenate([logits, jnp.zeros((pad,), jnp.float32)]).reshape(1, NP)
    seg_p = jnp.concatenate(
        [batch_non_omni, jnp.full((pad,), B, jnp.int32)]
    ).reshape(1, NP)
    u_p = jnp.concatenate([u, jnp.full((pad,), 0.5, jnp.float32)]).reshape(1, NP)

    lp_p, act2 = pl.pallas_call(
        _segment_body,
        in_specs=[
            pl.BlockSpec((1, NP), lambda: (0, 0)),
            pl.BlockSpec((1, NP), lambda: (0, 0)),
            pl.BlockSpec((1, NP), lambda: (0, 0)),
            pl.BlockSpec((B, 1), lambda: (0, 0)),
        ],
        out_specs=[
            pl.BlockSpec((1, NP), lambda: (0, 0)),
            pl.BlockSpec((B, 1), lambda: (0, 0)),
        ],
        out_shape=[
            jax.ShapeDtypeStruct((1, NP), jnp.float32),
            jax.ShapeDtypeStruct((B, 1), jnp.int32),
        ],
    )(lg_p, seg_p, u_p, act_offsets.reshape(B, 1))

    log_probs = lp_p.reshape(NP)[:n]
    act = act2.reshape(B)
    return (logits, log_probs, act)


# speedup vs baseline: 1.4874x; 1.1200x over previous
import jax
import jax.numpy as jnp
from jax.experimental import pallas as pl
from jax.experimental.pallas import tpu as pltpu

B = 256
NP = 10240
ROWS = 1280
NEG = -1e30
IMAX = 2147483647


def _body(
    e_ref, w1_ref, b1_ref, w2_ref, b2_ref, w3_ref, b3_ref, seg_ref, u_ref, off_ref,
    lg_ref, lp_ref, act_ref, scratch,
):
    i = pl.program_id(0)
    nsteps = pl.num_programs(0)

    eb = e_ref[...]
    h1 = jax.lax.dot_general(
        w1_ref[...], eb, (((0,), (1,)), ((), ())), preferred_element_type=jnp.float32
    )
    h1 = jnp.maximum(h1 + b1_ref[...], 0.0)
    h2 = jax.lax.dot_general(
        w2_ref[...], h1, (((0,), (0,)), ((), ())), preferred_element_type=jnp.float32
    )
    h2 = jnp.maximum(h2 + b2_ref[...], 0.0)
    lb = jax.lax.dot_general(
        w3_ref[...], h2, (((0,), (0,)), ((), ())), preferred_element_type=jnp.float32
    )
    lb = lb + b3_ref[...]
    scratch[:, pl.ds(i * ROWS, ROWS)] = lb

    @pl.when(i == nsteps - 1)
    def _segment():
        seg = seg_ref[...]
        lg = scratch[...]
        ids = jax.lax.broadcasted_iota(jnp.int32, (B, NP), 0)
        mask = seg == ids

        segmax = jnp.max(jnp.where(mask, lg, NEG), axis=1, keepdims=True)
        gmax_node = jnp.sum(jnp.where(mask, segmax, 0.0), axis=0, keepdims=True)
        shifted = lg - gmax_node

        segsum = jnp.sum(
            jnp.where(mask, jnp.exp(shifted), 0.0), axis=1, keepdims=True
        )
        logz = jnp.log(segsum)
        logz_node = jnp.sum(jnp.where(mask, logz, 0.0), axis=0, keepdims=True)
        lp = shifted - logz_node
        lg_ref[...] = lg
        lp_ref[...] = lp

        gum = -jnp.log(-jnp.log(u_ref[...]))
        gl = lp + gum
        gmax2 = jnp.max(jnp.where(mask, gl, NEG), axis=1, keepdims=True)
        idxs = jax.lax.broadcasted_iota(jnp.int32, (B, NP), 1)
        cand = jnp.where(mask & (gl == gmax2), idxs, jnp.int32(IMAX))
        arg = jnp.min(cand, axis=1, keepdims=True)
        act_ref[...] = arg - off_ref[...]


def kernel(e, u, batch_non_omni, act_offsets, W1, b1, W2, b2, W3, b3):
    n, k = e.shape
    h = W1.shape[1]
    pad = NP - n

    seg_p = jnp.concatenate(
        [batch_non_omni, jnp.full((pad,), B, jnp.int32)]
    ).reshape(1, NP)
    u_p = jnp.concatenate([u, jnp.full((pad,), 0.5, jnp.float32)]).reshape(1, NP)

    lg_p, lp_p, act2 = pl.pallas_call(
        _body,
        grid=(NP // ROWS,),
        in_specs=[
            pl.BlockSpec((ROWS, k), lambda i: (i, 0)),
            pl.BlockSpec((k, h), lambda i: (0, 0)),
            pl.BlockSpec((h, 1), lambda i: (0, 0)),
            pl.BlockSpec((h, h), lambda i: (0, 0)),
            pl.BlockSpec((h, 1), lambda i: (0, 0)),
            pl.BlockSpec((h, 1), lambda i: (0, 0)),
            pl.BlockSpec((1, 1), lambda i: (0, 0)),
            pl.BlockSpec((1, NP), lambda i: (0, 0)),
            pl.BlockSpec((1, NP), lambda i: (0, 0)),
            pl.BlockSpec((B, 1), lambda i: (0, 0)),
        ],
        out_specs=[
            pl.BlockSpec((1, NP), lambda i: (0, 0)),
            pl.BlockSpec((1, NP), lambda i: (0, 0)),
            pl.BlockSpec((B, 1), lambda i: (0, 0)),
        ],
        out_shape=[
            jax.ShapeDtypeStruct((1, NP), jnp.float32),
            jax.ShapeDtypeStruct((1, NP), jnp.float32),
            jax.ShapeDtypeStruct((B, 1), jnp.int32),
        ],
        scratch_shapes=[pltpu.VMEM((1, NP), jnp.float32)],
    )(
        e,
        W1,
        b1.reshape(h, 1),
        W2,
        b2.reshape(h, 1),
        W3,
        b3.reshape(1, 1),
        seg_p,
        u_p,
        act_offsets.reshape(B, 1),
    )

    logits = lg_p.reshape(NP)[:n]
    log_probs = lp_p.reshape(NP)[:n]
    act = act2.reshape(B)
    return (logits, log_probs, act)


# speedup vs baseline: 1.6013x; 1.0766x over previous
import jax
import jax.numpy as jnp
from jax.experimental import pallas as pl
from jax.experimental.pallas import tpu as pltpu

B = 256
NP = 10240
ROWS = 1280
NEG = -1e30
IMAX = 2147483647


def _body(
    e_ref, w1_ref, b1_ref, w2_ref, b2_ref, w3_ref, b3_ref, seg_ref, segf_ref,
    u_ref, off_ref,
    lg_ref, lp_ref, act_ref,
    lg_scr, m_scr, s_scr, amax_scr, arg_scr,
):
    i = pl.program_id(0)
    nsteps = pl.num_programs(0)

    eb = e_ref[...]
    h1 = jax.lax.dot_general(
        w1_ref[...], eb, (((0,), (1,)), ((), ())), preferred_element_type=jnp.float32
    )
    h1 = jnp.maximum(h1 + b1_ref[...], 0.0)
    h2 = jax.lax.dot_general(
        w2_ref[...], h1, (((0,), (0,)), ((), ())), preferred_element_type=jnp.float32
    )
    h2 = jnp.maximum(h2 + b2_ref[...], 0.0)
    lb = jax.lax.dot_general(
        w3_ref[...], h2, (((0,), (0,)), ((), ())), preferred_element_type=jnp.float32
    )
    lb = lb + b3_ref[...]
    lg_scr[:, pl.ds(i * ROWS, ROWS)] = lb

    @pl.when(i == 0)
    def _init():
        m_scr[...] = jnp.full((B, 1), NEG, jnp.float32)
        s_scr[...] = jnp.zeros((B, 1), jnp.float32)
        amax_scr[...] = jnp.full((B, 1), NEG, jnp.float32)
        arg_scr[...] = jnp.full((B, 1), IMAX, jnp.int32)

    segb = seg_ref[...]
    ids = jax.lax.broadcasted_iota(jnp.int32, (B, ROWS), 0)
    maskb = segb == ids

    m_old = m_scr[...]
    bmax = jnp.max(jnp.where(maskb, lb, NEG), axis=1, keepdims=True)
    m_new = jnp.maximum(m_old, bmax)
    bsum = jnp.sum(
        jnp.where(maskb, jnp.exp(lb - m_new), 0.0), axis=1, keepdims=True
    )
    s_scr[...] = s_scr[...] * jnp.exp(m_old - m_new) + bsum
    m_scr[...] = m_new

    gum = -jnp.log(-jnp.log(u_ref[...]))
    sb = lb + gum
    bamax = jnp.max(jnp.where(maskb, sb, NEG), axis=1, keepdims=True)
    bidx = jax.lax.broadcasted_iota(jnp.int32, (B, ROWS), 1) + i * ROWS
    barg = jnp.min(
        jnp.where(maskb & (sb == bamax), bidx, jnp.int32(IMAX)),
        axis=1,
        keepdims=True,
    )
    a_old = amax_scr[...]
    arg_old = arg_scr[...]
    amax_scr[...] = jnp.maximum(a_old, bamax)
    arg_scr[...] = jnp.where(
        bamax > a_old,
        barg,
        jnp.where(bamax == a_old, jnp.minimum(arg_old, barg), arg_old),
    )

    @pl.when(i == nsteps - 1)
    def _finish():
        s = s_scr[...]
        mlz = jnp.where(s > 0.0, m_scr[...] + jnp.log(s), 0.0)
        idsf = jax.lax.broadcasted_iota(jnp.int32, (B, NP), 0)
        maskf = (segf_ref[...] == idsf).astype(jnp.float32)
        mlz_node = jax.lax.dot_general(
            mlz, maskf, (((0,), (0,)), ((), ())), preferred_element_type=jnp.float32
        )
        lg = lg_scr[...]
        lg_ref[...] = lg
        lp_ref[...] = lg - mlz_node
        act_ref[...] = arg_scr[...] - off_ref[...]


def kernel(e, u, batch_non_omni, act_offsets, W1, b1, W2, b2, W3, b3):
    n, k = e.shape
    h = W1.shape[1]
    pad = NP - n

    seg_p = jnp.concatenate(
        [batch_non_omni, jnp.full((pad,), B, jnp.int32)]
    ).reshape(1, NP)
    u_p = jnp.concatenate([u, jnp.full((pad,), 0.5, jnp.float32)]).reshape(1, NP)

    lg_p, lp_p, act2 = pl.pallas_call(
        _body,
        grid=(NP // ROWS,),
        in_specs=[
            pl.BlockSpec((ROWS, k), lambda i: (i, 0)),
            pl.BlockSpec((k, h), lambda i: (0, 0)),
            pl.BlockSpec((h, 1), lambda i: (0, 0)),
            pl.BlockSpec((h, h), lambda i: (0, 0)),
            pl.BlockSpec((h, 1), lambda i: (0, 0)),
            pl.BlockSpec((h, 1), lambda i: (0, 0)),
            pl.BlockSpec((1, 1), lambda i: (0, 0)),
            pl.BlockSpec((1, ROWS), lambda i: (0, i)),
            pl.BlockSpec((1, NP), lambda i: (0, 0)),
            pl.BlockSpec((1, ROWS), lambda i: (0, i)),
            pl.BlockSpec((B, 1), lambda i: (0, 0)),
        ],
        out_specs=[
            pl.BlockSpec((1, NP), lambda i: (0, 0)),
            pl.BlockSpec((1, NP), lambda i: (0, 0)),
            pl.BlockSpec((B, 1), lambda i: (0, 0)),
        ],
        out_shape=[
            jax.ShapeDtypeStruct((1, NP), jnp.float32),
            jax.ShapeDtypeStruct((1, NP), jnp.float32),
            jax.ShapeDtypeStruct((B, 1), jnp.int32),
        ],
        scratch_shapes=[
            pltpu.VMEM((1, NP), jnp.float32),
            pltpu.VMEM((B, 1), jnp.float32),
            pltpu.VMEM((B, 1), jnp.float32),
            pltpu.VMEM((B, 1), jnp.float32),
            pltpu.VMEM((B, 1), jnp.int32),
        ],
    )(
        e,
        W1,
        b1.reshape(h, 1),
        W2,
        b2.reshape(h, 1),
        W3,
        b3.reshape(1, 1),
        seg_p,
        seg_p,
        u_p,
        act_offsets.reshape(B, 1),
    )

    logits = lg_p.reshape(NP)[:n]
    log_probs = lp_p.reshape(NP)[:n]
    act = act2.reshape(B)
    return (logits, log_probs, act)
